# baseline (speedup 1.0000x reference)
import functools

import jax
import jax.numpy as jnp
from jax.experimental import pallas as pl
from jax.experimental.pallas import tpu as pltpu

_VB = 8192
_CK = _VB // 128


def _fold(u_ref, l_ref, acc_y, acc_c, *, v, s, b, vocab, masked):
    u = u_ref[...]
    il = jnp.log(u)
    ol = jnp.log(-il)
    ay = acc_y[...]
    ac = acc_c[...]
    lane = jax.lax.broadcasted_iota(jnp.int32, (s, b, 128), 2)
    for c in range(_CK):
        y = ol[:, :, c * 128:(c + 1) * 128] - l_ref[:, c * 128:(c + 1) * 128][None]
        if masked:
            col = v * _VB + c * 128 + lane
            y = jnp.where(col < vocab, y, jnp.inf)
        cb = v * _CK + c
        take = y < ay
        ay = jnp.where(take, y, ay)
        ac = jnp.where(take, cb, ac)
    acc_y[...] = ay
    acc_c[...] = ac


def _body(u_ref, l_ref, out_ref, acc_y, acc_c, ridx, oh, *, nvb, vocab, s, b):
    p = pl.program_id(0)
    v = pl.program_id(1)

    @pl.when(jnp.logical_and(p == 0, v == 0))
    def _init():
        acc_y[...] = jnp.full((s, b, 128), jnp.inf, jnp.float32)
        acc_c[...] = jnp.zeros((s, b, 128), jnp.int32)

    @pl.when(jnp.logical_and(p == 0, v < nvb - 1))
    def _scan():
        _fold(u_ref, l_ref, acc_y, acc_c, v=v, s=s, b=b, vocab=vocab,
              masked=False)

    @pl.when(jnp.logical_and(p == 0, v == nvb - 1))
    def _scan_tail():
        _fold(u_ref, l_ref, acc_y, acc_c, v=v, s=s, b=b, vocab=vocab,
              masked=True)
        ay = acc_y[...]
        lane = jax.lax.broadcasted_iota(jnp.int32, (s, b, 128), 2)
        cols = acc_c[...] * 128 + lane
        gmin = jnp.min(ay, axis=-1)
        cand = jnp.where(ay == gmin[..., None], cols, jnp.iinfo(jnp.int32).max)
        r = jnp.min(cand, axis=-1)
        ridx[...] = r // 128
        oh[...] = (lane == (r - (r // 128) * 128)[..., None]).astype(
            jnp.float32)

    @pl.when(p == 1)
    def _write():
        rdiv = ridx[...]
        ohv = oh[...]
        for c in range(_CK):
            m = (rdiv == v * _CK + c).astype(jnp.float32)
            out_ref[:, :, c * 128:(c + 1) * 128] = ohv * m[..., None]


def kernel(logits, uniform_noise):
    s, b, vocab = uniform_noise.shape
    nvb = pl.cdiv(vocab, _VB)
    grid = (2, nvb)
    out = pl.pallas_call(
        functools.partial(_body, nvb=nvb, vocab=vocab, s=s, b=b),
        grid=grid,
        in_specs=[
            pl.BlockSpec(
                (s, b, _VB),
                lambda p, v: (0, 0, jnp.where(p == 0, v, nvb - 1)),
            ),
            pl.BlockSpec(
                (b, _VB),
                lambda p, v: (0, jnp.where(p == 0, v, nvb - 1)),
            ),
        ],
        out_specs=pl.BlockSpec(
            (s, b, _VB),
            lambda p, v: (0, 0, jnp.where(p == 0, 0, v)),
        ),
        out_shape=jax.ShapeDtypeStruct((s, b, vocab), jnp.float32),
        scratch_shapes=[
            pltpu.VMEM((s, b, 128), jnp.float32),
            pltpu.VMEM((s, b, 128), jnp.int32),
            pltpu.VMEM((s, b), jnp.int32),
            pltpu.VMEM((s, b, 128), jnp.float32),
        ],
        compiler_params=pltpu.CompilerParams(
            dimension_semantics=("arbitrary", "arbitrary"),
        ),
    )(uniform_noise, logits)
    return out


# speedup vs baseline: 1.0367x; 1.0367x over previous
import functools

import jax
import jax.numpy as jnp
from jax.experimental import pallas as pl
from jax.experimental.pallas import tpu as pltpu

_VBS = 8192
_CKS = _VBS // 128
_VBW = 16384


def _fold(u_ref, l_ref, acc_y, acc_c, *, v, s, b, vocab, masked):
    u = u_ref[...]
    il = jnp.log(u)
    ol = jnp.log(-il)
    ay = acc_y[...]
    ac = acc_c[...]
    lane = jax.lax.broadcasted_iota(jnp.int32, (s, b, 128), 2)
    for c in range(_CKS):
        y = ol[:, :, c * 128:(c + 1) * 128] - l_ref[:, c * 128:(c + 1) * 128][None]
        if masked:
            col = v * _VBS + c * 128 + lane
            y = jnp.where(col < vocab, y, jnp.inf)
        cb = v * _CKS + c
        take = y < ay
        ay = jnp.where(take, y, ay)
        ac = jnp.where(take, cb, ac)
    acc_y[...] = ay
    acc_c[...] = ac


def _scan_body(u_ref, l_ref, ridx_ref, acc_y, acc_c, *, nvb, vocab, s, b):
    v = pl.program_id(0)

    @pl.when(v == 0)
    def _init():
        acc_y[...] = jnp.full((s, b, 128), jnp.inf, jnp.float32)
        acc_c[...] = jnp.zeros((s, b, 128), jnp.int32)

    @pl.when(v < nvb - 1)
    def _scan():
        _fold(u_ref, l_ref, acc_y, acc_c, v=v, s=s, b=b, vocab=vocab,
              masked=False)

    @pl.when(v == nvb - 1)
    def _tail():
        _fold(u_ref, l_ref, acc_y, acc_c, v=v, s=s, b=b, vocab=vocab,
              masked=True)
        ay = acc_y[...]
        lane = jax.lax.broadcasted_iota(jnp.int32, (s, b, 128), 2)
        cols = acc_c[...] * 128 + lane
        gmin = jnp.min(ay, axis=-1)
        cand = jnp.where(ay == gmin[..., None], cols, jnp.iinfo(jnp.int32).max)
        ridx_ref[...] = jnp.min(cand, axis=-1)


def _write_body(ridx_ref, out_ref, *, s, b):
    v = pl.program_id(0)
    col = v * _VBW + jax.lax.broadcasted_iota(jnp.int32, (s, b, _VBW), 2)
    out_ref[...] = (col == ridx_ref[...][..., None]).astype(jnp.float32)


def kernel(logits, uniform_noise):
    s, b, vocab = uniform_noise.shape
    nvb = pl.cdiv(vocab, _VBS)
    ridx = pl.pallas_call(
        functools.partial(_scan_body, nvb=nvb, vocab=vocab, s=s, b=b),
        grid=(nvb,),
        in_specs=[
            pl.BlockSpec((s, b, _VBS), lambda v: (0, 0, v)),
            pl.BlockSpec((b, _VBS), lambda v: (0, v)),
        ],
        out_specs=pl.BlockSpec((s, b), lambda v: (0, 0)),
        out_shape=jax.ShapeDtypeStruct((s, b), jnp.int32),
        scratch_shapes=[
            pltpu.VMEM((s, b, 128), jnp.float32),
            pltpu.VMEM((s, b, 128), jnp.int32),
        ],
        compiler_params=pltpu.CompilerParams(
            dimension_semantics=("arbitrary",),
        ),
    )(uniform_noise, logits)
    nvw = pl.cdiv(vocab, _VBW)
    out = pl.pallas_call(
        functools.partial(_write_body, s=s, b=b),
        grid=(nvw,),
        in_specs=[pl.BlockSpec((s, b), lambda v: (0, 0))],
        out_specs=pl.BlockSpec((s, b, _VBW), lambda v: (0, 0, v)),
        out_shape=jax.ShapeDtypeStruct((s, b, vocab), jnp.float32),
        compiler_params=pltpu.CompilerParams(
            dimension_semantics=("arbitrary",),
        ),
    )(ridx)
    return out


# speedup vs baseline: 1.0636x; 1.0259x over previous
import functools

import jax
import jax.numpy as jnp
from jax.experimental import pallas as pl
from jax.experimental.pallas import tpu as pltpu

_VB = 8192
_CK = _VB // 128


def _fold(u_ref, l_ref, acc_y, acc_c, *, v, s, b, vocab, masked):
    u = u_ref[...]
    il = jnp.log(u)
    ol = jnp.log(-il)
    ay = acc_y[...]
    ac = acc_c[...]
    lane = jax.lax.broadcasted_iota(jnp.int32, (s, b, 128), 2)
    for c in range(_CK):
        y = ol[:, :, c * 128:(c + 1) * 128] - l_ref[:, c * 128:(c + 1) * 128][None]
        if masked:
            col = v * _VB + c * 128 + lane
            y = jnp.where(col < vocab, y, jnp.inf)
        cb = v * _CK + c
        take = y < ay
        ay = jnp.where(take, y, ay)
        ac = jnp.where(take, cb, ac)
    acc_y[...] = ay
    acc_c[...] = ac


def _body(u_ref, l_ref, out_ref, acc_y, acc_c, ridx, *, nvb, vocab, s, b):
    p = pl.program_id(0)
    v = pl.program_id(1)

    @pl.when(jnp.logical_and(p == 0, v == 0))
    def _init():
        acc_y[...] = jnp.full((s, b, 128), jnp.inf, jnp.float32)
        acc_c[...] = jnp.zeros((s, b, 128), jnp.int32)

    @pl.when(jnp.logical_and(p == 0, v < nvb - 1))
    def _scan():
        _fold(u_ref, l_ref, acc_y, acc_c, v=v, s=s, b=b, vocab=vocab,
              masked=False)

    @pl.when(jnp.logical_and(p == 0, v == nvb - 1))
    def _scan_tail():
        _fold(u_ref, l_ref, acc_y, acc_c, v=v, s=s, b=b, vocab=vocab,
              masked=True)
        ay = acc_y[...]
        lane = jax.lax.broadcasted_iota(jnp.int32, (s, b, 128), 2)
        cols = acc_c[...] * 128 + lane
        gmin = jnp.min(ay, axis=-1)
        cand = jnp.where(ay == gmin[..., None], cols, jnp.iinfo(jnp.int32).max)
        ridx[...] = jnp.min(cand, axis=-1)

    @pl.when(p == 1)
    def _write():
        col = v * _VB + jax.lax.broadcasted_iota(jnp.int32, (s, b, _VB), 2)
        out_ref[...] = (col == ridx[...][..., None]).astype(jnp.float32)


def kernel(logits, uniform_noise):
    s, b, vocab = uniform_noise.shape
    nvb = pl.cdiv(vocab, _VB)
    grid = (2, nvb)
    out = pl.pallas_call(
        functools.partial(_body, nvb=nvb, vocab=vocab, s=s, b=b),
        grid=grid,
        in_specs=[
            pl.BlockSpec(
                (s, b, _VB),
                lambda p, v: (0, 0, jnp.where(p == 0, v, nvb - 1)),
            ),
            pl.BlockSpec(
                (b, _VB),
                lambda p, v: (0, jnp.where(p == 0, v, nvb - 1)),
            ),
        ],
        out_specs=pl.BlockSpec(
            (s, b, _VB),
            lambda p, v: (0, 0, jnp.where(p == 0, 0, v)),
        ),
        out_shape=jax.ShapeDtypeStruct((s, b, vocab), jnp.float32),
        scratch_shapes=[
            pltpu.VMEM((s, b, 128), jnp.float32),
            pltpu.VMEM((s, b, 128), jnp.int32),
            pltpu.VMEM((s, b), jnp.int32),
        ],
        compiler_params=pltpu.CompilerParams(
            dimension_semantics=("arbitrary", "arbitrary"),
        ),
    )(uniform_noise, logits)
    return out
